# Initial kernel scaffold; baseline (speedup 1.0000x reference)
import jax
import jax.numpy as jnp
from jax.experimental import pallas as pl


def kernel(inputs, embeddings):
    raise NotImplementedError("write your pallas kernel here")


# speedup vs baseline: 9.2642x; 9.2642x over previous
import functools

import jax
import jax.numpy as jnp
from jax.experimental import pallas as pl
from jax.experimental.pallas import tpu as pltpu

_BN = 256
_BK = 8192


def _dist_kernel(x_ref, e_ref, o_ref):
    x = x_ref[...]
    e = e_ref[...]
    xx = jnp.sum(x * x, axis=1, keepdims=True)
    ee = jnp.sum(e * e, axis=1, keepdims=True).T
    xe = jax.lax.dot_general(
        x, e, (((1,), (1,)), ((), ())),
        preferred_element_type=jnp.float32)
    d2 = xx + ee - 2.0 * xe
    o_ref[...] = jnp.sqrt(jnp.maximum(d2, 0.0))


@functools.partial(jax.jit, static_argnames=())
def kernel(inputs, embeddings):
    n, d = inputs.shape
    k, _ = embeddings.shape
    grid = (n // _BN, k // _BK)
    return pl.pallas_call(
        _dist_kernel,
        grid=grid,
        in_specs=[
            pl.BlockSpec((_BN, d), lambda i, j: (i, 0)),
            pl.BlockSpec((_BK, d), lambda i, j: (j, 0)),
        ],
        out_specs=pl.BlockSpec((_BN, _BK), lambda i, j: (i, j)),
        out_shape=jax.ShapeDtypeStruct((n, k), jnp.float32),
        compiler_params=pltpu.CompilerParams(
            dimension_semantics=("arbitrary", "arbitrary"),
        ),
    )(inputs, embeddings)


# speedup vs baseline: 18.3502x; 1.9808x over previous
import functools

import jax
import jax.numpy as jnp
from jax.experimental import pallas as pl
from jax.experimental.pallas import tpu as pltpu

_BN = 256


def _dist_kernel(x_ref, e_ref, o_ref, ea_ref):
    bk = e_ref.shape[0]

    @pl.when(pl.program_id(0) == 0)
    def _build_e_aug():
        e = e_ref[...]
        ee = jnp.sum(e * e, axis=1, keepdims=True)
        ea_ref[...] = jnp.concatenate(
            [e, jnp.ones((bk, 1), jnp.float32), ee], axis=1)

    x = x_ref[...]
    xx = jnp.sum(x * x, axis=1, keepdims=True)
    xa = jnp.concatenate(
        [-2.0 * x, xx, jnp.ones((x.shape[0], 1), jnp.float32)], axis=1)
    d2 = jax.lax.dot_general(
        xa, ea_ref[...], (((1,), (1,)), ((), ())),
        preferred_element_type=jnp.float32)
    d2 = jnp.maximum(d2, 1e-36)
    o_ref[...] = d2 * jax.lax.rsqrt(d2)


@functools.partial(jax.jit, static_argnames=())
def kernel(inputs, embeddings):
    n, d = inputs.shape
    k, _ = embeddings.shape
    return pl.pallas_call(
        _dist_kernel,
        grid=(n // _BN,),
        in_specs=[
            pl.BlockSpec((_BN, d), lambda i: (i, 0)),
            pl.BlockSpec((k, d), lambda i: (0, 0)),
        ],
        out_specs=pl.BlockSpec((_BN, k), lambda i: (i, 0)),
        out_shape=jax.ShapeDtypeStruct((n, k), jnp.float32),
        scratch_shapes=[pltpu.VMEM((k, d + 2), jnp.float32)],
        compiler_params=pltpu.CompilerParams(
            dimension_semantics=("arbitrary",),
        ),
    )(inputs, embeddings)
